# Initial kernel scaffold; baseline (speedup 1.0000x reference)
import jax
import jax.numpy as jnp
from jax.experimental import pallas as pl


def kernel(x, edge_index, W1, b1, W2, b2):
    raise NotImplementedError("write your pallas kernel here")


# speedup vs baseline: 56.7791x; 56.7791x over previous
import functools

import jax
import jax.numpy as jnp
from jax import lax
from jax.experimental import pallas as pl
from jax.experimental.pallas import tpu as pltpu
from jax.experimental.pallas import tpu_sc as plsc

N = 10000
E = 320000
F_IN = 128
F_H = 64
NC, NS, LANES = 2, 16, 16
NW = NC * NS
EPW = E // NW
CHP = 128
KCH = EPW // CHP
TAIL = EPW - KCH * CHP
N_PAD = 10240
ROWS_PT = N_PAD // NS
BLK = 1000


def _sc_mesh():
    return plsc.VectorSubcoreMesh(
        core_axis_name="c", subcore_axis_name="s",
        num_cores=NC, num_subcores=NS)


_SC_PARAMS = pltpu.CompilerParams(use_tc_tiling_on_sc=False)


@functools.partial(
    pl.kernel,
    out_type=jax.ShapeDtypeStruct((N_PAD, 8 * LANES), jnp.float32),
    mesh=_sc_mesh(),
    compiler_params=_SC_PARAMS,
    scratch_types=(
        pltpu.VMEM((EPW,), jnp.int32),
        pltpu.VMEM((CHP, LANES), jnp.float32),
        pltpu.VMEM((64, LANES), jnp.float32),
        pltpu.VMEM_SHARED((N_PAD, LANES), jnp.float32),
        pltpu.SemaphoreType.DMA,
        pltpu.SemaphoreType.DMA,
        pltpu.SemaphoreType.DMA,
        pltpu.SemaphoreType.DMA,
        pltpu.SemaphoreType.DMA,
        pltpu.SemaphoreType.DMA,
    ),
)
def _deg_kernel(edges, out, dbuf, ones, zbuf, acc,
                sem, st, sd0, sd1, sd2, sd3):
    sem_d = (sd0, sd1, sd2, sd3)
    c = lax.axis_index("c")
    s = lax.axis_index("s")
    wid = c * NS + s

    pltpu.async_copy(edges.at[1, pl.ds(wid * EPW, EPW)], dbuf, sem)

    def fill(i, _):
        ones[i, pl.ds(0, LANES)] = jnp.ones((LANES,), jnp.float32)
        return 0
    lax.fori_loop(0, CHP, fill, 0)

    def zero(i, _):
        zbuf[i, pl.ds(0, LANES)] = jnp.zeros((LANES,), jnp.float32)
        return 0
    lax.fori_loop(0, 64, zero, 0)

    row0 = s * ROWS_PT

    def zacc(t, _):
        pltpu.sync_copy(zbuf, acc.at[pl.ds(row0 + 64 * t, 64)])
        return 0
    lax.fori_loop(0, ROWS_PT // 64, zacc, 0)
    pltpu.make_async_copy(edges.at[1, pl.ds(wid * EPW, EPW)], dbuf, sem).wait()
    plsc.subcore_barrier()

    def didx(j):
        return dbuf.at[pl.ds(j * CHP, CHP)]

    tidx = dbuf.at[pl.ds(KCH * CHP, TAIL)]
    pltpu.async_copy(ones.at[pl.ds(0, TAIL)], acc.at[tidx], st, add=True)

    def body(i, _):
        for b in range(4):
            j = 4 * i + b

            @pl.when(j >= 4)
            def _():
                pltpu.make_async_copy(ones, acc.at[didx(j - 4)],
                                      sem_d[b]).wait()

            @pl.when(j < KCH)
            def _():
                pltpu.async_copy(ones, acc.at[didx(j)], sem_d[b], add=True)
        return 0
    lax.fori_loop(0, (KCH + 3) // 4, body, 0)
    for b in range(2):
        pltpu.make_async_copy(ones, acc.at[didx(KCH - 2 + b)],
                              sem_d[(KCH - 2 + b) % 4]).wait()
    pltpu.make_async_copy(ones.at[pl.ds(0, TAIL)], acc.at[tidx], st).wait()
    plsc.subcore_barrier()

    @pl.when(c == 0)
    def _():
        pltpu.sync_copy(acc.at[pl.ds(row0, ROWS_PT)],
                        out.at[pl.ds(row0, ROWS_PT), pl.ds(0, LANES)])

    @pl.when(c == 1)
    def _():
        pltpu.sync_copy(acc.at[pl.ds(row0, ROWS_PT)],
                        out.at[pl.ds(row0, ROWS_PT), pl.ds(LANES, LANES)])


@functools.partial(
    pl.kernel,
    out_type=jax.ShapeDtypeStruct((N_PAD, 2 * F_H), jnp.float32),
    mesh=_sc_mesh(),
    compiler_params=_SC_PARAMS,
    scratch_types=(
        pltpu.VMEM((EPW,), jnp.int32),
        pltpu.VMEM((EPW,), jnp.int32),
        pltpu.VMEM((CHP, F_H), jnp.float32),
        pltpu.VMEM((CHP, F_H), jnp.float32),
        pltpu.VMEM((CHP, F_H), jnp.float32),
        pltpu.VMEM((CHP, F_H), jnp.float32),
        pltpu.VMEM((CHP, F_H), jnp.float32),
        pltpu.VMEM((CHP, F_H), jnp.float32),
        pltpu.VMEM((TAIL, F_H), jnp.float32),
        pltpu.VMEM((64, F_H), jnp.float32),
        pltpu.VMEM_SHARED((N_PAD, F_H), jnp.float32),
        pltpu.SemaphoreType.DMA,
        pltpu.SemaphoreType.DMA,
        pltpu.SemaphoreType.DMA,
        pltpu.SemaphoreType.DMA,
        pltpu.SemaphoreType.DMA,
        pltpu.SemaphoreType.DMA,
        pltpu.SemaphoreType.DMA,
        pltpu.SemaphoreType.DMA,
        pltpu.SemaphoreType.DMA,
        pltpu.SemaphoreType.DMA,
        pltpu.SemaphoreType.DMA,
        pltpu.SemaphoreType.DMA,
        pltpu.SemaphoreType.DMA,
        pltpu.SemaphoreType.DMA,
        pltpu.SemaphoreType.DMA,
    ),
)
def _prop_kernel(hs, edges, out,
                 sbuf, dbuf, msg0, msg1, msg2, msg3, msg4, msg5, msgt, zbuf,
                 acc, sg0, sg1, sg2, sg3, sg4, sg5,
                 ss0, ss1, ss2, ss3, ss4, ss5, stg, sts, sem_i):
    msg = (msg0, msg1, msg2, msg3, msg4, msg5)
    sem_g = (sg0, sg1, sg2, sg3, sg4, sg5)
    sem_s = (ss0, ss1, ss2, ss3, ss4, ss5)
    c = lax.axis_index("c")
    s = lax.axis_index("s")
    wid = c * NS + s
    row0 = s * ROWS_PT

    eblk = pl.ds(wid * EPW, EPW)
    pltpu.async_copy(edges.at[0, eblk], sbuf, sem_i)
    pltpu.async_copy(edges.at[1, eblk], dbuf, sem_i)

    def zero(i, _):
        for k in range(F_H // LANES):
            zbuf[i, pl.ds(k * LANES, LANES)] = jnp.zeros((LANES,), jnp.float32)
        return 0
    lax.fori_loop(0, 64, zero, 0)

    def zacc(t, _):
        for b in range(5):
            tt = 5 * t + b
            pltpu.async_copy(zbuf, acc.at[pl.ds(row0 + 64 * tt, 64)],
                             sem_s[b])
        return 0
    lax.fori_loop(0, ROWS_PT // 64 // 5, zacc, 0)

    def zacc_drain(t, _):
        for b in range(5):
            tt = 5 * t + b
            pltpu.make_async_copy(zbuf, acc.at[pl.ds(row0 + 64 * tt, 64)],
                                  sem_s[b]).wait()
        return 0
    lax.fori_loop(0, ROWS_PT // 64 // 5, zacc_drain, 0)

    pltpu.make_async_copy(edges.at[0, eblk], sbuf, sem_i).wait()
    pltpu.make_async_copy(edges.at[1, eblk], dbuf, sem_i).wait()
    plsc.subcore_barrier()

    def gather(j, b):
        pltpu.async_copy(hs.at[sbuf.at[pl.ds(j * CHP, CHP)]], msg[b], sem_g[b])

    def wait_gather(j, b):
        pltpu.make_async_copy(hs.at[sbuf.at[pl.ds(j * CHP, CHP)]], msg[b],
                              sem_g[b]).wait()

    def scatter(j, b):
        pltpu.async_copy(msg[b], acc.at[dbuf.at[pl.ds(j * CHP, CHP)]],
                         sem_s[b], add=True)

    def drain_scatter(j, b):
        pltpu.make_async_copy(msg[b], acc.at[dbuf.at[pl.ds(j * CHP, CHP)]],
                              sem_s[b]).wait()

    tsl = pl.ds(KCH * CHP, TAIL)
    pltpu.async_copy(hs.at[sbuf.at[tsl]], msgt, stg)

    gather(0, 0)
    gather(1, 1)
    gather(2, 2)

    def body(i, _):
        for b in range(6):
            j = 6 * i + b

            @pl.when(j >= 3)
            def _():
                drain_scatter(j - 3, (b + 3) % 6)

            @pl.when(j + 3 < KCH)
            def _():
                gather(j + 3, (b + 3) % 6)

            wait_gather(j, b)
            scatter(j, b)
        return 0
    lax.fori_loop(0, KCH // 6, body, 0)
    pltpu.make_async_copy(hs.at[sbuf.at[tsl]], msgt, stg).wait()
    pltpu.async_copy(msgt, acc.at[dbuf.at[tsl]], sts, add=True)
    for j in (KCH - 3, KCH - 2, KCH - 1):
        drain_scatter(j, j % 6)
    pltpu.make_async_copy(msgt, acc.at[dbuf.at[tsl]], sts).wait()
    plsc.subcore_barrier()

    @pl.when(c == 0)
    def _():
        pltpu.sync_copy(acc.at[pl.ds(row0, ROWS_PT)],
                        out.at[pl.ds(row0, ROWS_PT), pl.ds(0, F_H)])

    @pl.when(c == 1)
    def _():
        pltpu.sync_copy(acc.at[pl.ds(row0, ROWS_PT)],
                        out.at[pl.ds(row0, ROWS_PT), pl.ds(F_H, F_H)])


def _mm1s_body(x_ref, w_ref, d_ref, hs_ref, dinv_ref):
    deg = d_ref[:, 0:1] + d_ref[:, LANES:LANES + 1] + 1.0
    dinv = lax.rsqrt(deg)
    dinv_ref[...] = dinv
    hs_ref[...] = jnp.dot(x_ref[...], w_ref[...],
                          preferred_element_type=jnp.float32) * dinv


def _mm1s(x, w1, d):
    return pl.pallas_call(
        _mm1s_body,
        grid=(N // BLK,),
        in_specs=[pl.BlockSpec((BLK, F_IN), lambda i: (i, 0)),
                  pl.BlockSpec((F_IN, F_H), lambda i: (0, 0)),
                  pl.BlockSpec((BLK, 8 * LANES), lambda i: (i, 0))],
        out_specs=[pl.BlockSpec((BLK, F_H), lambda i: (i, 0)),
                   pl.BlockSpec((BLK, 1), lambda i: (i, 0))],
        out_shape=[jax.ShapeDtypeStruct((N, F_H), jnp.float32),
                   jax.ShapeDtypeStruct((N, 1), jnp.float32)],
    )(x, w1, d)


def _mid_body(p_ref, hs_ref, dinv_ref, b_ref, w_ref, o_ref):
    dinv = dinv_ref[...]
    p = p_ref[:, :F_H] + p_ref[:, F_H:]
    t = dinv * (p + hs_ref[...]) + b_ref[...]
    t = jnp.maximum(t, 0.0)
    o_ref[...] = jnp.dot(t, w_ref[...],
                         preferred_element_type=jnp.float32) * dinv


def _mid(p, hs1, dinv, b1, w2):
    return pl.pallas_call(
        _mid_body,
        grid=(N // BLK,),
        in_specs=[pl.BlockSpec((BLK, 2 * F_H), lambda i: (i, 0)),
                  pl.BlockSpec((BLK, F_H), lambda i: (i, 0)),
                  pl.BlockSpec((BLK, 1), lambda i: (i, 0)),
                  pl.BlockSpec((1, F_H), lambda i: (0, 0)),
                  pl.BlockSpec((F_H, F_H), lambda i: (0, 0))],
        out_specs=pl.BlockSpec((BLK, F_H), lambda i: (i, 0)),
        out_shape=jax.ShapeDtypeStruct((N, F_H), jnp.float32),
    )(p, hs1, dinv, b1, w2)


def _final_body(p_ref, hs_ref, dinv_ref, b_ref, o_ref):
    dinv = dinv_ref[...]
    p = p_ref[:, :F_H] + p_ref[:, F_H:]
    t = dinv * (p + hs_ref[...]) + b_ref[...]
    t = jnp.maximum(t, 0.0)
    part = jnp.sum(t, axis=0, keepdims=True) * (1.0 / N)

    @pl.when(pl.program_id(0) == 0)
    def _():
        o_ref[...] = jnp.zeros_like(o_ref)

    o_ref[...] += part


def _final(q, hs2, dinv, b2):
    return pl.pallas_call(
        _final_body,
        grid=(N // BLK,),
        in_specs=[pl.BlockSpec((BLK, 2 * F_H), lambda i: (i, 0)),
                  pl.BlockSpec((BLK, F_H), lambda i: (i, 0)),
                  pl.BlockSpec((BLK, 1), lambda i: (i, 0)),
                  pl.BlockSpec((1, F_H), lambda i: (0, 0))],
        out_specs=pl.BlockSpec((1, F_H), lambda i: (0, 0)),
        out_shape=jax.ShapeDtypeStruct((1, F_H), jnp.float32),
    )(q, hs2, dinv, b2)


def kernel(x, edge_index, W1, b1, W2, b2):
    d = _deg_kernel(edge_index)
    hs1, dinv = _mm1s(x, W1, d)
    p = _prop_kernel(hs1, edge_index)
    hs2 = _mid(p, hs1, dinv, b1.reshape(1, F_H), W2)
    q = _prop_kernel(hs2, edge_index)
    return _final(q, hs2, dinv, b2.reshape(1, F_H))
